# Initial kernel scaffold; baseline (speedup 1.0000x reference)
import jax
import jax.numpy as jnp
from jax.experimental import pallas as pl


def kernel(x, edge_index, W1, b1, W2, b2, temp):
    raise NotImplementedError("write your pallas kernel here")


# speedup vs baseline: 14.5086x; 14.5086x over previous
import functools
import math

import jax
import jax.numpy as jnp
import numpy as np
from jax import lax
from jax.experimental import pallas as pl
from jax.experimental.pallas import tpu as pltpu
from jax.experimental.pallas import tpu_sc as plsc

N = 10000
E = 320000
D_IN = 128
HID = 64
F = 40
K = 10

NW = 32
NSUB = 16
CH = 128
NCH = 79
EW = NCH * CH
EPAD = NW * EW
NP = 10240
RPT = NP // NSUB


def _cheb_nodes_matrix(k):
    xs = [math.cos((k - j + 0.5) * math.pi / (k + 1)) for j in range(k + 1)]
    rows = []
    for i in range(k + 1):
        rows.append([math.cos(i * math.acos(max(-1.0, min(1.0, xj)))) for xj in xs])
    return rows


_CHEB_M = np.asarray(_cheb_nodes_matrix(K), dtype=np.float32)

@functools.cache
def _make_deg_sc():
    mesh = plsc.VectorSubcoreMesh(core_axis_name="c", subcore_axis_name="s")
    return pl.kernel(
        _deg_body,
        out_type=jax.ShapeDtypeStruct((2, NP), jnp.float32),
        mesh=mesh,
        scratch_types=[
            pltpu.VMEM((NCH, CH), jnp.int32),
            pltpu.VMEM((CH,), jnp.float32),
            pltpu.VMEM_SHARED((NP,), jnp.float32),
        ],
        compiler_params=pltpu.CompilerParams(use_tc_tiling_on_sc=False),
    )


def _deg_body(src_hbm, zeros_hbm, out_hbm, sidx_v, ones_v, acc_sh):
    cid = lax.axis_index("c")
    sid = lax.axis_index("s")
    wid = cid * NSUB + sid
    pltpu.sync_copy(src_hbm.at[wid], sidx_v)
    for i in range(CH // 16):
        ones_v[pl.ds(16 * i, 16)] = jnp.full((16,), 1.0, jnp.float32)
    pltpu.sync_copy(zeros_hbm, acc_sh.at[pl.ds(sid * RPT, RPT)])
    plsc.subcore_barrier()

    def body(j, carry):
        pltpu.sync_copy(ones_v, acc_sh.at[sidx_v.at[j]], add=True)
        return carry

    lax.fori_loop(0, NCH, body, 0)
    plsc.subcore_barrier()
    pltpu.sync_copy(acc_sh.at[pl.ds(sid * RPT, RPT)],
                    out_hbm.at[cid, pl.ds(sid * RPT, RPT)])


@functools.cache
def _make_prop_sc():
    mesh = plsc.VectorSubcoreMesh(core_axis_name="c", subcore_axis_name="s")
    return pl.kernel(
        _prop_body,
        out_type=jax.ShapeDtypeStruct((2, NP, F), jnp.float32),
        mesh=mesh,
        scratch_types=[
            pltpu.VMEM((NCH, CH), jnp.int32),
            pltpu.VMEM((NCH, CH), jnp.int32),
            pltpu.VMEM((CH, F), jnp.float32),
            pltpu.VMEM_SHARED((NP, F), jnp.float32),
            pltpu.SemaphoreType.DMA,
        ],
        compiler_params=pltpu.CompilerParams(use_tc_tiling_on_sc=False),
    )


def _prop_body(u_hbm, src_hbm, dst_hbm, zeros_hbm, out_hbm,
               sidx_v, didx_v, rows_v, acc_sh, sem):
    cid = lax.axis_index("c")
    sid = lax.axis_index("s")
    wid = cid * NSUB + sid
    pltpu.sync_copy(src_hbm.at[wid], sidx_v)
    pltpu.sync_copy(dst_hbm.at[wid], didx_v)
    pltpu.sync_copy(zeros_hbm, acc_sh.at[pl.ds(sid * RPT, RPT)])
    plsc.subcore_barrier()

    def body(j, carry):
        pltpu.async_copy(u_hbm.at[sidx_v.at[j]], rows_v, sem).wait()
        pltpu.sync_copy(rows_v, acc_sh.at[didx_v.at[j]], add=True)
        return carry

    lax.fori_loop(0, NCH, body, 0)
    plsc.subcore_barrier()
    pltpu.sync_copy(acc_sh.at[pl.ds(sid * RPT, RPT)],
                    out_hbm.at[cid, pl.ds(sid * RPT, RPT)])


def _mlp_body(coe_ref, x_ref, w1_ref, b1_ref, w2_ref, b2_ref, deg_ref,
              h_ref, u0_ref, dis_ref, out0_ref):
    h1 = jnp.maximum(
        jnp.dot(x_ref[...], w1_ref[...].T, preferred_element_type=jnp.float32)
        + b1_ref[...][None, :], 0.0)
    h = (jnp.dot(h1, w2_ref[...].T, preferred_element_type=jnp.float32)
         + b2_ref[...][None, :])
    deg = deg_ref[...]
    dis = jnp.where(deg > 0.0, lax.rsqrt(jnp.maximum(deg, 1.0)), 0.0)
    h_ref[...] = h
    u0_ref[...] = h * dis
    dis_ref[...] = dis
    out0_ref[...] = (0.5 * coe_ref[0]) * h


_mlp_tc = pl.pallas_call(
    _mlp_body,
    out_shape=(
        jax.ShapeDtypeStruct((NP, F), jnp.float32),
        jax.ShapeDtypeStruct((NP, F), jnp.float32),
        jax.ShapeDtypeStruct((NP, 1), jnp.float32),
        jax.ShapeDtypeStruct((NP, F), jnp.float32),
    ),
    in_specs=[pl.BlockSpec(memory_space=pltpu.SMEM)]
    + [pl.BlockSpec(memory_space=pltpu.VMEM)] * 6,
)


def _step_body(sc_ref, s_ref, tx0_ref, out_ref, dis_ref,
               tx2_ref, outn_ref, u_ref):
    s = s_ref[0] + s_ref[1]
    dis = dis_ref[...]
    tx2 = sc_ref[0] * dis * s + sc_ref[1] * tx0_ref[...]
    tx2_ref[...] = tx2
    outn_ref[...] = out_ref[...] + sc_ref[2] * tx2
    u_ref[...] = dis * tx2


_step_tc = pl.pallas_call(
    _step_body,
    out_shape=(
        jax.ShapeDtypeStruct((NP, F), jnp.float32),
        jax.ShapeDtypeStruct((NP, F), jnp.float32),
        jax.ShapeDtypeStruct((NP, F), jnp.float32),
    ),
    in_specs=[pl.BlockSpec(memory_space=pltpu.SMEM)]
    + [pl.BlockSpec(memory_space=pltpu.VMEM)] * 4,
)


def _lsm_body(x_ref, o_ref):
    x = x_ref[...]
    m = jnp.max(x, axis=1, keepdims=True)
    e = jnp.exp(x - m)
    o_ref[...] = (x - m) - jnp.log(jnp.sum(e, axis=1, keepdims=True))


_lsm_tc = pl.pallas_call(
    _lsm_body,
    out_shape=jax.ShapeDtypeStruct((N, F), jnp.float32),
)


def kernel(x, edge_index, W1, b1, W2, b2, temp):
    src = edge_index[0]
    dst = edge_index[1]
    pad = EPAD - E
    padidx = (N + (jnp.arange(pad, dtype=jnp.int32) % CH)).astype(jnp.int32)
    srcp = jnp.concatenate([src, padidx]).reshape(NW, NCH, CH)
    dstp = jnp.concatenate([dst, padidx]).reshape(NW, NCH, CH)
    xp = jnp.pad(x, ((0, NP - N), (0, 0)))
    zrows = jnp.zeros((RPT, F), jnp.float32)
    zdeg = jnp.zeros((RPT,), jnp.float32)

    coe = (2.0 / (K + 1)) * (_CHEB_M @ jnp.maximum(temp, 0.0))

    degs = _make_deg_sc()(srcp, zdeg)
    deg = (degs[0] + degs[1]).reshape(NP, 1)

    h, u, dis, out = _mlp_tc(coe, xp, W1, b1, W2, b2, deg)

    tx0 = h
    tx1 = h
    for i in range(1, K + 1):
        s = _make_prop_sc()(u, srcp, dstp, zrows)
        a = jnp.float32(-1.0 if i == 1 else -2.0)
        b = jnp.float32(0.0 if i == 1 else -1.0)
        sc = jnp.stack([a, b, coe[i]])
        tx2, out, u = _step_tc(sc, s, tx0, out, dis)
        tx0, tx1 = tx1, tx2

    return _lsm_tc(out[:N])


# speedup vs baseline: 20.7275x; 1.4286x over previous
import functools
import math

import jax
import jax.numpy as jnp
import numpy as np
from jax import lax
from jax.experimental import pallas as pl
from jax.experimental.pallas import tpu as pltpu
from jax.experimental.pallas import tpu_sc as plsc

N = 10000
E = 320000
D_IN = 128
HID = 64
F = 40
K = 10

NW = 32
NSUB = 16
CH = 128
NCH = 80
EW = NCH * CH
EPAD = NW * EW
NP = 10240
RPT = NP // NSUB


def _cheb_nodes_matrix(k):
    xs = [math.cos((k - j + 0.5) * math.pi / (k + 1)) for j in range(k + 1)]
    rows = []
    for i in range(k + 1):
        rows.append([math.cos(i * math.acos(max(-1.0, min(1.0, xj)))) for xj in xs])
    return rows


_CHEB_M = np.asarray(_cheb_nodes_matrix(K), dtype=np.float32)

@functools.cache
def _make_deg_sc():
    mesh = plsc.VectorSubcoreMesh(core_axis_name="c", subcore_axis_name="s")
    return pl.kernel(
        _deg_body,
        out_type=jax.ShapeDtypeStruct((2, NP), jnp.float32),
        mesh=mesh,
        scratch_types=[
            pltpu.VMEM((NCH, CH), jnp.int32),
            pltpu.VMEM((CH,), jnp.float32),
            pltpu.VMEM_SHARED((NP,), jnp.float32),
        ],
        compiler_params=pltpu.CompilerParams(use_tc_tiling_on_sc=False),
    )


def _deg_body(src_hbm, zeros_hbm, out_hbm, sidx_v, ones_v, acc_sh):
    cid = lax.axis_index("c")
    sid = lax.axis_index("s")
    wid = cid * NSUB + sid
    pltpu.sync_copy(src_hbm.at[wid], sidx_v)
    for i in range(CH // 16):
        ones_v[pl.ds(16 * i, 16)] = jnp.full((16,), 1.0, jnp.float32)
    pltpu.sync_copy(zeros_hbm, acc_sh.at[pl.ds(sid * RPT, RPT)])
    plsc.subcore_barrier()

    def body(j, carry):
        pltpu.sync_copy(ones_v, acc_sh.at[sidx_v.at[j]], add=True)
        return carry

    lax.fori_loop(0, NCH, body, 0)
    plsc.subcore_barrier()
    pltpu.sync_copy(acc_sh.at[pl.ds(sid * RPT, RPT)],
                    out_hbm.at[cid, pl.ds(sid * RPT, RPT)])


@functools.cache
def _make_prop_sc():
    mesh = plsc.VectorSubcoreMesh(core_axis_name="c", subcore_axis_name="s")
    return pl.kernel(
        _prop_body,
        out_type=jax.ShapeDtypeStruct((2, NP, F), jnp.float32),
        mesh=mesh,
        scratch_types=[
            pltpu.VMEM((NCH, CH), jnp.int32),
            pltpu.VMEM((NCH, CH), jnp.int32),
            pltpu.VMEM((CH, F), jnp.float32),
            pltpu.VMEM((CH, F), jnp.float32),
            pltpu.VMEM_SHARED((NP, F), jnp.float32),
            pltpu.SemaphoreType.DMA,
            pltpu.SemaphoreType.DMA,
        ],
        compiler_params=pltpu.CompilerParams(use_tc_tiling_on_sc=False),
    )


def _prop_body(u_hbm, src_hbm, dst_hbm, zeros_hbm, out_hbm,
               sidx_v, didx_v, r0_v, r1_v, acc_sh, sem0, sem1):
    cid = lax.axis_index("c")
    sid = lax.axis_index("s")
    wid = cid * NSUB + sid
    bufs = (r0_v, r1_v)
    sems = (sem0, sem1)
    pltpu.sync_copy(src_hbm.at[wid], sidx_v)
    pltpu.sync_copy(dst_hbm.at[wid], didx_v)
    pltpu.sync_copy(zeros_hbm, acc_sh.at[pl.ds(sid * RPT, RPT)])
    plsc.subcore_barrier()

    pltpu.async_copy(u_hbm.at[sidx_v.at[0]], r0_v, sem0)

    def body(g, carry):
        for b in range(2):
            j = 2 * g + b
            pltpu.async_copy(u_hbm.at[sidx_v.at[j + 1]], bufs[1 - b],
                             sems[1 - b])
            pltpu.make_async_copy(u_hbm.at[sidx_v.at[j]], bufs[b],
                                  sems[b]).wait()
            pltpu.sync_copy(bufs[b], acc_sh.at[didx_v.at[j]], add=True)
        return carry

    lax.fori_loop(0, (NCH - 2) // 2, body, 0)
    pltpu.async_copy(u_hbm.at[sidx_v.at[NCH - 1]], r1_v, sem1)
    pltpu.make_async_copy(u_hbm.at[sidx_v.at[NCH - 2]], r0_v, sem0).wait()
    pltpu.sync_copy(r0_v, acc_sh.at[didx_v.at[NCH - 2]], add=True)
    pltpu.make_async_copy(u_hbm.at[sidx_v.at[NCH - 1]], r1_v, sem1).wait()
    pltpu.sync_copy(r1_v, acc_sh.at[didx_v.at[NCH - 1]], add=True)
    plsc.subcore_barrier()
    pltpu.sync_copy(acc_sh.at[pl.ds(sid * RPT, RPT)],
                    out_hbm.at[cid, pl.ds(sid * RPT, RPT)])


def _mlp_body(coe_ref, x_ref, w1_ref, b1_ref, w2_ref, b2_ref, deg_ref,
              h_ref, u0_ref, dis_ref, out0_ref):
    h1 = jnp.maximum(
        jnp.dot(x_ref[...], w1_ref[...].T, preferred_element_type=jnp.float32)
        + b1_ref[...][None, :], 0.0)
    h = (jnp.dot(h1, w2_ref[...].T, preferred_element_type=jnp.float32)
         + b2_ref[...][None, :])
    deg = deg_ref[...]
    dis = jnp.where(deg > 0.0, lax.rsqrt(jnp.maximum(deg, 1.0)), 0.0)
    h_ref[...] = h
    u0_ref[...] = h * dis
    dis_ref[...] = dis
    out0_ref[...] = (0.5 * coe_ref[0]) * h


_mlp_tc = pl.pallas_call(
    _mlp_body,
    out_shape=(
        jax.ShapeDtypeStruct((NP, F), jnp.float32),
        jax.ShapeDtypeStruct((NP, F), jnp.float32),
        jax.ShapeDtypeStruct((NP, 1), jnp.float32),
        jax.ShapeDtypeStruct((NP, F), jnp.float32),
    ),
    in_specs=[pl.BlockSpec(memory_space=pltpu.SMEM)]
    + [pl.BlockSpec(memory_space=pltpu.VMEM)] * 6,
)


def _step_body(sc_ref, s_ref, tx0_ref, out_ref, dis_ref,
               tx2_ref, outn_ref, u_ref):
    s = s_ref[0] + s_ref[1]
    dis = dis_ref[...]
    tx2 = sc_ref[0] * dis * s + sc_ref[1] * tx0_ref[...]
    tx2_ref[...] = tx2
    outn_ref[...] = out_ref[...] + sc_ref[2] * tx2
    u_ref[...] = dis * tx2


_step_tc = pl.pallas_call(
    _step_body,
    out_shape=(
        jax.ShapeDtypeStruct((NP, F), jnp.float32),
        jax.ShapeDtypeStruct((NP, F), jnp.float32),
        jax.ShapeDtypeStruct((NP, F), jnp.float32),
    ),
    in_specs=[pl.BlockSpec(memory_space=pltpu.SMEM)]
    + [pl.BlockSpec(memory_space=pltpu.VMEM)] * 4,
)


def _lsm_body(x_ref, o_ref):
    x = x_ref[...]
    m = jnp.max(x, axis=1, keepdims=True)
    e = jnp.exp(x - m)
    o_ref[...] = (x - m) - jnp.log(jnp.sum(e, axis=1, keepdims=True))


_lsm_tc = pl.pallas_call(
    _lsm_body,
    out_shape=jax.ShapeDtypeStruct((N, F), jnp.float32),
)


def kernel(x, edge_index, W1, b1, W2, b2, temp):
    src = edge_index[0]
    dst = edge_index[1]
    pad = EPAD - E
    padidx = (N + (jnp.arange(pad, dtype=jnp.int32) % CH)).astype(jnp.int32)
    srcp = jnp.concatenate([src, padidx]).reshape(NW, NCH, CH)
    dstp = jnp.concatenate([dst, padidx]).reshape(NW, NCH, CH)
    xp = jnp.pad(x, ((0, NP - N), (0, 0)))
    zrows = jnp.zeros((RPT, F), jnp.float32)
    zdeg = jnp.zeros((RPT,), jnp.float32)

    coe = (2.0 / (K + 1)) * (_CHEB_M @ jnp.maximum(temp, 0.0))

    degs = _make_deg_sc()(srcp, zdeg)
    deg = (degs[0] + degs[1]).reshape(NP, 1)

    h, u, dis, out = _mlp_tc(coe, xp, W1, b1, W2, b2, deg)

    tx0 = h
    tx1 = h
    for i in range(1, K + 1):
        s = _make_prop_sc()(u, srcp, dstp, zrows)
        a = jnp.float32(-1.0 if i == 1 else -2.0)
        b = jnp.float32(0.0 if i == 1 else -1.0)
        sc = jnp.stack([a, b, coe[i]])
        tx2, out, u = _step_tc(sc, s, tx0, out, dis)
        tx0, tx1 = tx1, tx2

    return _lsm_tc(out[:N])


# speedup vs baseline: 25.8646x; 1.2478x over previous
import functools
import math

import jax
import jax.numpy as jnp
import numpy as np
from jax import lax
from jax.experimental import pallas as pl
from jax.experimental.pallas import tpu as pltpu
from jax.experimental.pallas import tpu_sc as plsc

N = 10000
E = 320000
D_IN = 128
HID = 64
F = 40
K = 10

NW = 32
NSUB = 16
CH = 128
NCH = 80
EW = NCH * CH
EPAD = NW * EW
RB = 8
NP = 10240
RPT = NP // NSUB


def _cheb_nodes_matrix(k):
    xs = [math.cos((k - j + 0.5) * math.pi / (k + 1)) for j in range(k + 1)]
    rows = []
    for i in range(k + 1):
        rows.append([math.cos(i * math.acos(max(-1.0, min(1.0, xj)))) for xj in xs])
    return rows


_CHEB_M = np.asarray(_cheb_nodes_matrix(K), dtype=np.float32)

@functools.cache
def _make_deg_sc():
    mesh = plsc.VectorSubcoreMesh(core_axis_name="c", subcore_axis_name="s")
    return pl.kernel(
        _deg_body,
        out_type=jax.ShapeDtypeStruct((2, NP), jnp.float32),
        mesh=mesh,
        scratch_types=[
            pltpu.VMEM((NCH, CH), jnp.int32),
            pltpu.VMEM((CH,), jnp.float32),
            pltpu.VMEM_SHARED((NP,), jnp.float32),
            pltpu.SemaphoreType.DMA,
        ],
        compiler_params=pltpu.CompilerParams(use_tc_tiling_on_sc=False),
    )


def _deg_body(src_hbm, zeros_hbm, out_hbm, sidx_v, ones_v, acc_sh, sem):
    cid = lax.axis_index("c")
    sid = lax.axis_index("s")
    wid = cid * NSUB + sid
    pltpu.sync_copy(src_hbm.at[wid], sidx_v)
    for i in range(CH // 16):
        ones_v[pl.ds(16 * i, 16)] = jnp.full((16,), 1.0, jnp.float32)
    pltpu.sync_copy(zeros_hbm, acc_sh.at[pl.ds(sid * RPT, RPT)])
    plsc.subcore_barrier()

    def fire(j, carry):
        pltpu.async_copy(ones_v, acc_sh.at[sidx_v.at[j]], sem, add=True)
        return carry

    lax.fori_loop(0, NCH, fire, 0)

    def drain(j, carry):
        pltpu.make_async_copy(ones_v, acc_sh.at[sidx_v.at[j]], sem).wait()
        return carry

    lax.fori_loop(0, NCH, drain, 0)
    plsc.subcore_barrier()
    pltpu.sync_copy(acc_sh.at[pl.ds(sid * RPT, RPT)],
                    out_hbm.at[cid, pl.ds(sid * RPT, RPT)])


@functools.cache
def _make_prop_sc():
    mesh = plsc.VectorSubcoreMesh(core_axis_name="c", subcore_axis_name="s")
    return pl.kernel(
        _prop_body,
        out_type=jax.ShapeDtypeStruct((2, NP, F), jnp.float32),
        mesh=mesh,
        scratch_types=[
            pltpu.VMEM((NCH, CH), jnp.int32),
            pltpu.VMEM((NCH, CH), jnp.int32),
            [pltpu.VMEM((CH, F), jnp.float32)] * RB,
            [pltpu.SemaphoreType.DMA] * RB,
            [pltpu.SemaphoreType.DMA] * RB,
            pltpu.VMEM_SHARED((NP, F), jnp.float32),
        ],
        compiler_params=pltpu.CompilerParams(use_tc_tiling_on_sc=False),
    )


def _prop_body(u_hbm, src_hbm, dst_hbm, zeros_hbm, out_hbm,
               sidx_v, didx_v, bufs, gsems, ssems, acc_sh):
    cid = lax.axis_index("c")
    sid = lax.axis_index("s")
    wid = cid * NSUB + sid
    pltpu.sync_copy(src_hbm.at[wid], sidx_v)
    pltpu.sync_copy(dst_hbm.at[wid], didx_v)
    pltpu.sync_copy(zeros_hbm, acc_sh.at[pl.ds(sid * RPT, RPT)])
    plsc.subcore_barrier()

    HL = RB // 2

    def start_gather(j, b):
        pltpu.async_copy(u_hbm.at[sidx_v.at[j]], bufs[b], gsems[b])

    def wait_gather(j, b):
        pltpu.make_async_copy(u_hbm.at[sidx_v.at[j]], bufs[b],
                              gsems[b]).wait()

    def start_scatter(j, b):
        pltpu.async_copy(bufs[b], acc_sh.at[didx_v.at[j]], ssems[b],
                         add=True)

    def wait_scatter(j, b):
        pltpu.make_async_copy(bufs[b], acc_sh.at[didx_v.at[j]],
                              ssems[b]).wait()

    for b in range(HL):
        start_gather(b, b)
    for j in range(HL):
        start_gather(j + HL, j + HL)
        wait_gather(j, j)
        start_scatter(j, j)

    def body(g, carry):
        for b in range(RB):
            j = RB * g + b + HL
            wait_scatter(j - HL, b)
            start_gather(j + HL, b)
            wait_gather(j, (b + HL) % RB)
            start_scatter(j, (b + HL) % RB)
        return carry

    lax.fori_loop(0, (NCH - 2 * HL) // RB, body, 0)
    for j in range(NCH - HL, NCH):
        wait_scatter(j - HL, (j + HL) % RB)
        wait_gather(j, j % RB)
        start_scatter(j, j % RB)
    for j in range(NCH - HL, NCH):
        wait_scatter(j, j % RB)
    plsc.subcore_barrier()
    pltpu.sync_copy(acc_sh.at[pl.ds(sid * RPT, RPT)],
                    out_hbm.at[cid, pl.ds(sid * RPT, RPT)])


def _mlp_body(coe_ref, x_ref, w1_ref, b1_ref, w2_ref, b2_ref, deg_ref,
              h_ref, u0_ref, dis_ref, out0_ref):
    h1 = jnp.maximum(
        jnp.dot(x_ref[...], w1_ref[...].T, preferred_element_type=jnp.float32)
        + b1_ref[...][None, :], 0.0)
    h = (jnp.dot(h1, w2_ref[...].T, preferred_element_type=jnp.float32)
         + b2_ref[...][None, :])
    deg = deg_ref[...]
    dis = jnp.where(deg > 0.0, lax.rsqrt(jnp.maximum(deg, 1.0)), 0.0)
    h_ref[...] = h
    u0_ref[...] = h * dis
    dis_ref[...] = dis
    out0_ref[...] = (0.5 * coe_ref[0]) * h


_mlp_tc = pl.pallas_call(
    _mlp_body,
    out_shape=(
        jax.ShapeDtypeStruct((NP, F), jnp.float32),
        jax.ShapeDtypeStruct((NP, F), jnp.float32),
        jax.ShapeDtypeStruct((NP, 1), jnp.float32),
        jax.ShapeDtypeStruct((NP, F), jnp.float32),
    ),
    in_specs=[pl.BlockSpec(memory_space=pltpu.SMEM)]
    + [pl.BlockSpec(memory_space=pltpu.VMEM)] * 6,
)


def _step_body(sc_ref, s_ref, tx0_ref, out_ref, dis_ref,
               tx2_ref, outn_ref, u_ref):
    s = s_ref[0] + s_ref[1]
    dis = dis_ref[...]
    tx2 = sc_ref[0] * dis * s + sc_ref[1] * tx0_ref[...]
    tx2_ref[...] = tx2
    outn_ref[...] = out_ref[...] + sc_ref[2] * tx2
    u_ref[...] = dis * tx2


_step_tc = pl.pallas_call(
    _step_body,
    out_shape=(
        jax.ShapeDtypeStruct((NP, F), jnp.float32),
        jax.ShapeDtypeStruct((NP, F), jnp.float32),
        jax.ShapeDtypeStruct((NP, F), jnp.float32),
    ),
    in_specs=[pl.BlockSpec(memory_space=pltpu.SMEM)]
    + [pl.BlockSpec(memory_space=pltpu.VMEM)] * 4,
)


def _lsm_body(x_ref, o_ref):
    x = x_ref[...]
    m = jnp.max(x, axis=1, keepdims=True)
    e = jnp.exp(x - m)
    o_ref[...] = (x - m) - jnp.log(jnp.sum(e, axis=1, keepdims=True))


_lsm_tc = pl.pallas_call(
    _lsm_body,
    out_shape=jax.ShapeDtypeStruct((N, F), jnp.float32),
)


def kernel(x, edge_index, W1, b1, W2, b2, temp):
    src = edge_index[0]
    dst = edge_index[1]
    pad = EPAD - E
    padidx = (N + (jnp.arange(pad, dtype=jnp.int32) % CH)).astype(jnp.int32)
    srcp = jnp.concatenate([src, padidx]).reshape(NW, NCH, CH)
    dstp = jnp.concatenate([dst, padidx]).reshape(NW, NCH, CH)
    xp = jnp.pad(x, ((0, NP - N), (0, 0)))
    zrows = jnp.zeros((RPT, F), jnp.float32)
    zdeg = jnp.zeros((RPT,), jnp.float32)

    coe = (2.0 / (K + 1)) * (_CHEB_M @ jnp.maximum(temp, 0.0))

    degs = _make_deg_sc()(srcp, zdeg)
    deg = (degs[0] + degs[1]).reshape(NP, 1)

    h, u, dis, out = _mlp_tc(coe, xp, W1, b1, W2, b2, deg)

    tx0 = h
    tx1 = h
    for i in range(1, K + 1):
        s = _make_prop_sc()(u, srcp, dstp, zrows)
        a = jnp.float32(-1.0 if i == 1 else -2.0)
        b = jnp.float32(0.0 if i == 1 else -1.0)
        sc = jnp.stack([a, b, coe[i]])
        tx2, out, u = _step_tc(sc, s, tx0, out, dis)
        tx0, tx1 = tx1, tx2

    return _lsm_tc(out[:N])
